# baseline (speedup 1.0000x reference)
import jax
import jax.numpy as jnp
from jax import lax
from jax.experimental import pallas as pl
from jax.experimental.pallas import tpu as pltpu
from jax.experimental.pallas import tpu_sc as plsc

_HASH_VOCAB = 1000000
_BIGRAM_DIM = 64
_MODEL_DIM = 1024
_BOS_ID = 1

_NC = 2
_NS = 16
_NW = _NC * _NS
_LANES = 16

_B = 4
_S = 8192
_TOTAL = _B * _S
_CHUNK = _TOTAL // _NW

_VR = 31232
_SLAB = 256
_NSLAB = _VR // _SLAB
_TAIL_START = _HASH_VOCAB - 1024
_NTAIL = 4
_OUT_ROWS = _TOTAL + 128

_SENTINEL = 0x7F000000


def _hash_body(ids_hbm, out_hbm, ids_ext, hash_v, sem):
    del sem
    cid = lax.axis_index("c")
    sid = lax.axis_index("s")
    wid = cid * _NS + sid
    base = wid * _CHUNK

    ids_ext[pl.ds(0, _LANES)] = jnp.full((_LANES,), _BOS_ID, dtype=jnp.int32)
    pltpu.sync_copy(ids_hbm.at[pl.ds(base, _CHUNK)], ids_ext.at[pl.ds(8, _CHUNK)])

    @pl.when(lax.rem(wid, _S // _CHUNK) != 0)
    def _():
        pltpu.sync_copy(ids_hbm.at[pl.ds(base - 8, 8)], ids_ext.at[pl.ds(0, 8)])

    lanes = lax.iota(jnp.int32, _LANES)
    for j in range(8):
        for k in range(8):
            i = j * 8 + k
            cur = ids_ext[pl.ds(8 + i * _LANES, _LANES)]
            prev = plsc.load_gather(ids_ext, [lanes + (7 + i * _LANES)])
            x = (prev << 8) + (prev << 2) + (prev << 1) + prev + cur
            for c in (8000000, 4000000, 2000000, 1000000):
                x = jnp.where(x >= c, x - c, x)
            hash_v[j, pl.ds(k * _LANES, _LANES)] = x

    for j in range(8):
        pltpu.sync_copy(hash_v.at[j], out_hbm.at[pl.ds(base + j * 128, 128)])


def _sc_hash(ids_flat):
    mesh = plsc.VectorSubcoreMesh(core_axis_name="c", subcore_axis_name="s")
    return pl.kernel(
        _hash_body,
        out_type=jax.ShapeDtypeStruct((_TOTAL,), jnp.int32),
        mesh=mesh,
        scratch_types=[
            pltpu.VMEM((_CHUNK + 8,), jnp.int32),
            pltpu.VMEM((8, 128), jnp.int32),
            pltpu.SemaphoreType.DMA,
        ],
        compiler_params=pltpu.CompilerParams(needs_layout_passes=False),
    )(ids_flat)


def _gather_body(
    hash_hbm, embT_hbm, tail_hbm, out_hbm,
    hashchunk, binv, slablist, slab2d, stage, idxbuf, bkvec, sem0, sem1,
):
    cid = lax.axis_index("c")
    sid = lax.axis_index("s")
    wid = cid * _NS + sid
    v0 = wid * _VR
    lanes = lax.iota(jnp.int32, _LANES)
    is_last = wid == _NW - 1

    def _reset_idxbuf():
        for r in range(8):
            idxbuf[pl.ds(r * _LANES, _LANES)] = (
                lanes + (_TOTAL + r * _LANES)
            )

    _reset_idxbuf()

    def zrow(r, c):
        rv = jnp.full((_LANES,), r, jnp.int32)
        for g in range(4):
            plsc.store_scatter(
                stage,
                [rv, 64 + g * _LANES + lanes],
                jnp.zeros((_LANES,), jnp.float32),
            )
        return c

    lax.fori_loop(0, 128, zrow, 0)

    limit = jnp.where(is_last, jnp.int32(1 << 20), jnp.int32(_VR))

    def chunk_body(c, cnt):
        pltpu.sync_copy(hash_hbm.at[pl.ds(c * 4096, 4096)], hashchunk)

        def vreg_body(j, cnt):
            h = hashchunk[pl.ds(j * _LANES, _LANES)]
            hl = h - v0
            mask = (hl >= 0) & (hl < limit)
            nm = jnp.sum(mask.astype(jnp.int32))

            @pl.when(nm > 0)
            def _():
                pos = lanes + (c * 4096 + j * _LANES)
                key = (hl << 15) | pos
                plsc.store_compressed(
                    binv.at[pl.ds(cnt, _LANES)], key, mask=mask
                )

            return cnt + nm

        return lax.fori_loop(0, 4096 // _LANES, vreg_body, cnt)

    cnt = lax.fori_loop(0, _TOTAL // 4096, chunk_body, 0)
    plsc.store_compressed(
        binv.at[pl.ds(cnt, _LANES)],
        jnp.full((_LANES,), _SENTINEL, jnp.int32),
        mask=jnp.ones((_LANES,), jnp.bool_),
    )
    nv = (cnt + _LANES - 1) // _LANES

    _NBK = 8

    starts = [jnp.int32(0)]
    for i in range(_NBK):

        def bpass(v, cur, _i=i):
            k = binv[pl.ds(v * _LANES, _LANES)]
            m = (k >> 27) == _i
            plsc.store_compressed(
                slablist.at[pl.ds(cur, _LANES)], k, mask=m
            )
            return cur + jnp.sum(m.astype(jnp.int32))

        starts.append(lax.fori_loop(0, nv, bpass, starts[i]))
    plsc.store_compressed(
        slablist.at[pl.ds(cnt, _LANES)],
        jnp.full((_LANES,), _SENTINEL, jnp.int32),
        mask=jnp.ones((_LANES,), jnp.bool_),
    )
    bv = jnp.zeros((_LANES,), jnp.int32)
    for i in range(_NBK + 1):
        bv = jnp.where(lanes == i, starts[i], bv)
    bkvec[pl.ds(0, _LANES)] = bv

    nslab = jnp.where(is_last, jnp.int32(_NSLAB + _NTAIL), jnp.int32(_NSLAB))

    def fire(s, sem, slot):
        @pl.when(s < _NSLAB)
        def _():
            pltpu.async_copy(
                embT_hbm.at[:, pl.ds(v0 + s * _SLAB, _SLAB)],
                slab2d.at[pl.ds(slot * _BIGRAM_DIM, _BIGRAM_DIM), :],
                sem,
            )

        @pl.when(s >= _NSLAB)
        def _():
            pltpu.async_copy(
                tail_hbm.at[:, pl.ds((s - _NSLAB) * _SLAB, _SLAB)],
                slab2d.at[pl.ds(slot * _BIGRAM_DIM, _BIGRAM_DIM), :],
                sem,
            )

    def wait(sem, slot):
        pltpu.make_async_copy(
            embT_hbm.at[:, pl.ds(0, _SLAB)],
            slab2d.at[pl.ds(slot * _BIGRAM_DIM, _BIGRAM_DIM), :],
            sem,
        ).wait()

    def process(s, slot, tot):
        hl0 = jnp.where(
            s < _NSLAB,
            s * _SLAB,
            (_TAIL_START - v0) + (s - _NSLAB) * _SLAB,
        ).astype(jnp.int32)
        lo = hl0 << 15
        hi = (hl0 + _SLAB) << 15
        bkt = hl0 >> 12
        bvv = bkvec[pl.ds(0, _LANES)]
        start_b = jnp.sum(jnp.where(lanes == bkt, bvv, 0))
        end_b = jnp.sum(jnp.where(lanes == bkt + 1, bvv, 0))

        v0r = start_b // _LANES
        nvr = (end_b + _LANES - 1) // _LANES - v0r

        def scan(i, mcnt):
            k = slablist[pl.ds((v0r + i) * _LANES, _LANES)]
            m = (k >= lo) & (k < hi)
            plsc.store_compressed(binv.at[pl.ds(mcnt, _LANES)], k, mask=m)
            return mcnt + jnp.sum(m.astype(jnp.int32))

        mcnt = lax.fori_loop(0, nvr, scan, 0)

        def ex(m, tot):
            mb = (m // _LANES) * _LANES
            kv = binv[pl.ds(mb, _LANES)]
            key_v = jnp.take(
                kv,
                jnp.full((_LANES,), m - mb, jnp.int32),
                mode="fill",
            )
            col_v = (key_v >> 15) - hl0
            p_v = key_v & 32767
            r = lax.rem(tot, 128)
            rv = jnp.full((_LANES,), r, jnp.int32)
            for g in range(4):
                vals = plsc.load_gather(
                    slab2d,
                    [slot * _BIGRAM_DIM + g * _LANES + lanes, col_v],
                )
                plsc.store_scatter(stage, [rv, g * _LANES + lanes], vals)
            plsc.store_scatter(idxbuf, [rv], p_v, mask=lanes == 0)

            @pl.when(r == 127)
            def _():
                pltpu.sync_copy(stage, out_hbm.at[idxbuf])
                _reset_idxbuf()

            return tot + 1

        return lax.fori_loop(0, mcnt, ex, tot)

    fire(0, sem0, 0)

    def pair(i, tot):
        s0 = 2 * i
        fire(s0 + 1, sem1, 1)
        wait(sem0, 0)
        tot = process(s0, 0, tot)

        @pl.when(s0 + 2 < nslab)
        def _():
            fire(s0 + 2, sem0, 0)

        wait(sem1, 1)
        return process(s0 + 1, 1, tot)

    tot = lax.fori_loop(0, nslab // 2, pair, 0)

    @pl.when(lax.rem(tot, 128) != 0)
    def _():
        pltpu.sync_copy(stage, out_hbm.at[idxbuf])


def _sc_gather(hashes, embT, tail):
    mesh = plsc.VectorSubcoreMesh(core_axis_name="c", subcore_axis_name="s")
    return pl.kernel(
        _gather_body,
        out_type=jax.ShapeDtypeStruct((_OUT_ROWS, 128), jnp.float32),
        mesh=mesh,
        scratch_types=[
            pltpu.VMEM((4096,), jnp.int32),
            pltpu.VMEM((_TOTAL + _LANES,), jnp.int32),
            pltpu.VMEM((_TOTAL + _LANES,), jnp.int32),
            pltpu.VMEM((2 * _BIGRAM_DIM, _SLAB), jnp.float32),
            pltpu.VMEM((128, 128), jnp.float32),
            pltpu.VMEM((128,), jnp.int32),
            pltpu.VMEM((_LANES,), jnp.int32),
            pltpu.SemaphoreType.DMA,
            pltpu.SemaphoreType.DMA,
        ],
        compiler_params=pltpu.CompilerParams(needs_layout_passes=False),
    )(hashes, embT, tail)


_MM_ROWS = 2048


def _mm_body(emb_ref, w_ref, out_ref):
    out_ref[...] = jnp.dot(
        emb_ref[...], w_ref[...], preferred_element_type=jnp.float32
    )


def _tc_matmul(emb_pad, W_pad):
    return pl.pallas_call(
        _mm_body,
        grid=(_TOTAL // _MM_ROWS,),
        in_specs=[
            pl.BlockSpec((_MM_ROWS, 128), lambda i: (i, 0)),
            pl.BlockSpec((128, _MODEL_DIM), lambda i: (0, 0)),
        ],
        out_specs=pl.BlockSpec((_MM_ROWS, _MODEL_DIM), lambda i: (i, 0)),
        out_shape=jax.ShapeDtypeStruct((_TOTAL, _MODEL_DIM), jnp.float32),
    )(emb_pad, W_pad)


def kernel(ids, embed, W):
    ids_flat = ids.reshape(_TOTAL).astype(jnp.int32)
    embT = embed.T
    tail = lax.slice(embT, (0, _TAIL_START), (_BIGRAM_DIM, _HASH_VOCAB))
    hashes = _sc_hash(ids_flat)
    emb_pad = _sc_gather(hashes, embT, tail)
    W_pad = jnp.pad(W, ((0, 128 - _BIGRAM_DIM), (0, 0)))
    out = _tc_matmul(emb_pad, W_pad)
    return out.reshape(_B, _S, _MODEL_DIM)


# speedup vs baseline: 1.0789x; 1.0789x over previous
import jax
import jax.numpy as jnp
from jax import lax
from jax.experimental import pallas as pl
from jax.experimental.pallas import tpu as pltpu
from jax.experimental.pallas import tpu_sc as plsc

_HASH_VOCAB = 1000000
_BIGRAM_DIM = 64
_MODEL_DIM = 1024
_BOS_ID = 1

_NC = 2
_NS = 16
_NW = _NC * _NS
_LANES = 16

_B = 4
_S = 8192
_TOTAL = _B * _S
_CHUNK = _TOTAL // _NW

_VR = 31232
_SLAB = 256
_NSLAB = _VR // _SLAB
_TAIL_START = _HASH_VOCAB - 1024
_NTAIL = 4
_OUT_ROWS = _TOTAL + 128

_SENTINEL = 0x7F000000


def _hash_body(ids_hbm, out_hbm, ids_ext, hash_v, sem):
    del sem
    cid = lax.axis_index("c")
    sid = lax.axis_index("s")
    wid = cid * _NS + sid
    base = wid * _CHUNK

    ids_ext[pl.ds(0, _LANES)] = jnp.full((_LANES,), _BOS_ID, dtype=jnp.int32)
    pltpu.sync_copy(ids_hbm.at[pl.ds(base, _CHUNK)], ids_ext.at[pl.ds(8, _CHUNK)])

    @pl.when(lax.rem(wid, _S // _CHUNK) != 0)
    def _():
        pltpu.sync_copy(ids_hbm.at[pl.ds(base - 8, 8)], ids_ext.at[pl.ds(0, 8)])

    lanes = lax.iota(jnp.int32, _LANES)
    for j in range(8):
        for k in range(8):
            i = j * 8 + k
            cur = ids_ext[pl.ds(8 + i * _LANES, _LANES)]
            prev = plsc.load_gather(ids_ext, [lanes + (7 + i * _LANES)])
            x = (prev << 8) + (prev << 2) + (prev << 1) + prev + cur
            for c in (8000000, 4000000, 2000000, 1000000):
                x = jnp.where(x >= c, x - c, x)
            hash_v[j, pl.ds(k * _LANES, _LANES)] = x

    for j in range(8):
        pltpu.sync_copy(hash_v.at[j], out_hbm.at[pl.ds(base + j * 128, 128)])


def _sc_hash(ids_flat):
    mesh = plsc.VectorSubcoreMesh(core_axis_name="c", subcore_axis_name="s")
    return pl.kernel(
        _hash_body,
        out_type=jax.ShapeDtypeStruct((_TOTAL,), jnp.int32),
        mesh=mesh,
        scratch_types=[
            pltpu.VMEM((_CHUNK + 8,), jnp.int32),
            pltpu.VMEM((8, 128), jnp.int32),
            pltpu.SemaphoreType.DMA,
        ],
        compiler_params=pltpu.CompilerParams(needs_layout_passes=False),
    )(ids_flat)


def _gather_body(
    hash_hbm, embT_hbm, tail_hbm, out_hbm,
    hashchunk, binv, slablist, slab2d, stage, idxbuf, bkvec, sem0, sem1,
):
    cid = lax.axis_index("c")
    sid = lax.axis_index("s")
    wid = cid * _NS + sid
    v0 = wid * _VR
    lanes = lax.iota(jnp.int32, _LANES)
    is_last = wid == _NW - 1

    def _reset_idxbuf():
        for r in range(8):
            idxbuf[pl.ds(r * _LANES, _LANES)] = (
                lanes + (_TOTAL + r * _LANES)
            )

    _reset_idxbuf()

    def zrow(r, c):
        rv = jnp.full((_LANES,), r, jnp.int32)
        for g in range(4):
            plsc.store_scatter(
                stage,
                [rv, 64 + g * _LANES + lanes],
                jnp.zeros((_LANES,), jnp.float32),
            )
        return c

    lax.fori_loop(0, 128, zrow, 0)

    limit = jnp.where(is_last, jnp.int32(1 << 20), jnp.int32(_VR))

    def chunk_body(c, cnt):
        pltpu.sync_copy(hash_hbm.at[pl.ds(c * 4096, 4096)], hashchunk)

        def vreg_body(j, cnt):
            for u in range(2):
                h = hashchunk[pl.ds((2 * j + u) * _LANES, _LANES)]
                pos = lanes + (c * 4096 + (2 * j + u) * _LANES)
                hl = h - v0
                mask = (hl >= 0) & (hl < limit)
                key = (hl << 15) | pos
                plsc.store_compressed(
                    binv.at[pl.ds(cnt, _LANES)], key, mask=mask
                )
                cnt = cnt + jnp.sum(mask.astype(jnp.int32))
            return cnt

        return lax.fori_loop(0, 4096 // _LANES // 2, vreg_body, cnt)

    cnt = lax.fori_loop(0, _TOTAL // 4096, chunk_body, 0)
    plsc.store_compressed(
        binv.at[pl.ds(cnt, _LANES)],
        jnp.full((_LANES,), _SENTINEL, jnp.int32),
        mask=jnp.ones((_LANES,), jnp.bool_),
    )
    nv = (cnt + _LANES - 1) // _LANES

    _NBK = 8

    starts = [jnp.int32(0)]
    for i in range(_NBK):

        def bpass(v, cur, _i=i):
            k = binv[pl.ds(v * _LANES, _LANES)]
            m = (k >> 27) == _i
            plsc.store_compressed(
                slablist.at[pl.ds(cur, _LANES)], k, mask=m
            )
            return cur + jnp.sum(m.astype(jnp.int32))

        starts.append(lax.fori_loop(0, nv, bpass, starts[i]))
    plsc.store_compressed(
        slablist.at[pl.ds(cnt, _LANES)],
        jnp.full((_LANES,), _SENTINEL, jnp.int32),
        mask=jnp.ones((_LANES,), jnp.bool_),
    )
    bv = jnp.zeros((_LANES,), jnp.int32)
    for i in range(_NBK + 1):
        bv = jnp.where(lanes == i, starts[i], bv)
    bkvec[pl.ds(0, _LANES)] = bv

    nslab = jnp.where(is_last, jnp.int32(_NSLAB + _NTAIL), jnp.int32(_NSLAB))

    def fire(s, sem, slot):
        @pl.when(s < _NSLAB)
        def _():
            pltpu.async_copy(
                embT_hbm.at[:, pl.ds(v0 + s * _SLAB, _SLAB)],
                slab2d.at[pl.ds(slot * _BIGRAM_DIM, _BIGRAM_DIM), :],
                sem,
            )

        @pl.when(s >= _NSLAB)
        def _():
            pltpu.async_copy(
                tail_hbm.at[:, pl.ds((s - _NSLAB) * _SLAB, _SLAB)],
                slab2d.at[pl.ds(slot * _BIGRAM_DIM, _BIGRAM_DIM), :],
                sem,
            )

    def wait(sem, slot):
        pltpu.make_async_copy(
            embT_hbm.at[:, pl.ds(0, _SLAB)],
            slab2d.at[pl.ds(slot * _BIGRAM_DIM, _BIGRAM_DIM), :],
            sem,
        ).wait()

    def process(s, slot, tot):
        hl0 = jnp.where(
            s < _NSLAB,
            s * _SLAB,
            (_TAIL_START - v0) + (s - _NSLAB) * _SLAB,
        ).astype(jnp.int32)
        lo = hl0 << 15
        hi = (hl0 + _SLAB) << 15
        bkt = hl0 >> 12
        bvv = bkvec[pl.ds(0, _LANES)]
        start_b = jnp.sum(jnp.where(lanes == bkt, bvv, 0))
        end_b = jnp.sum(jnp.where(lanes == bkt + 1, bvv, 0))

        v0r = start_b // _LANES
        nvr = (end_b + _LANES - 1) // _LANES - v0r

        def scan(i, mcnt):
            k = slablist[pl.ds((v0r + i) * _LANES, _LANES)]
            m = (k >= lo) & (k < hi)
            plsc.store_compressed(binv.at[pl.ds(mcnt, _LANES)], k, mask=m)
            return mcnt + jnp.sum(m.astype(jnp.int32))

        mcnt = lax.fori_loop(0, nvr, scan, 0)

        def ex(m, tot):
            mb = (m // _LANES) * _LANES
            kv = binv[pl.ds(mb, _LANES)]
            key_v = jnp.take(
                kv,
                jnp.full((_LANES,), m - mb, jnp.int32),
                mode="fill",
            )
            col_v = (key_v >> 15) - hl0
            p_v = key_v & 32767
            r = lax.rem(tot, 128)
            rv = jnp.full((_LANES,), r, jnp.int32)
            for g in range(4):
                vals = plsc.load_gather(
                    slab2d,
                    [slot * _BIGRAM_DIM + g * _LANES + lanes, col_v],
                )
                plsc.store_scatter(stage, [rv, g * _LANES + lanes], vals)
            plsc.store_scatter(idxbuf, [rv], p_v, mask=lanes == 0)

            @pl.when(r == 127)
            def _():
                pltpu.sync_copy(stage, out_hbm.at[idxbuf])
                _reset_idxbuf()

            return tot + 1

        return lax.fori_loop(0, mcnt, ex, tot)

    fire(0, sem0, 0)

    def pair(i, tot):
        s0 = 2 * i
        fire(s0 + 1, sem1, 1)
        wait(sem0, 0)
        tot = process(s0, 0, tot)

        @pl.when(s0 + 2 < nslab)
        def _():
            fire(s0 + 2, sem0, 0)

        wait(sem1, 1)
        return process(s0 + 1, 1, tot)

    tot = lax.fori_loop(0, nslab // 2, pair, 0)

    @pl.when(lax.rem(tot, 128) != 0)
    def _():
        pltpu.sync_copy(stage, out_hbm.at[idxbuf])


def _sc_gather(hashes, embT, tail):
    mesh = plsc.VectorSubcoreMesh(core_axis_name="c", subcore_axis_name="s")
    return pl.kernel(
        _gather_body,
        out_type=jax.ShapeDtypeStruct((_OUT_ROWS, 128), jnp.float32),
        mesh=mesh,
        scratch_types=[
            pltpu.VMEM((4096,), jnp.int32),
            pltpu.VMEM((_TOTAL + _LANES,), jnp.int32),
            pltpu.VMEM((_TOTAL + _LANES,), jnp.int32),
            pltpu.VMEM((2 * _BIGRAM_DIM, _SLAB), jnp.float32),
            pltpu.VMEM((128, 128), jnp.float32),
            pltpu.VMEM((128,), jnp.int32),
            pltpu.VMEM((_LANES,), jnp.int32),
            pltpu.SemaphoreType.DMA,
            pltpu.SemaphoreType.DMA,
        ],
        compiler_params=pltpu.CompilerParams(needs_layout_passes=False),
    )(hashes, embT, tail)


_MM_ROWS = 2048


def _mm_body(emb_ref, w_ref, out_ref):
    out_ref[...] = jnp.dot(
        emb_ref[...], w_ref[...], preferred_element_type=jnp.float32
    )


def _tc_matmul(emb_pad, W_pad):
    return pl.pallas_call(
        _mm_body,
        grid=(_TOTAL // _MM_ROWS,),
        in_specs=[
            pl.BlockSpec((_MM_ROWS, 128), lambda i: (i, 0)),
            pl.BlockSpec((128, _MODEL_DIM), lambda i: (0, 0)),
        ],
        out_specs=pl.BlockSpec((_MM_ROWS, _MODEL_DIM), lambda i: (i, 0)),
        out_shape=jax.ShapeDtypeStruct((_TOTAL, _MODEL_DIM), jnp.float32),
    )(emb_pad, W_pad)


def kernel(ids, embed, W):
    ids_flat = ids.reshape(_TOTAL).astype(jnp.int32)
    embT = embed.T
    tail = lax.slice(embT, (0, _TAIL_START), (_BIGRAM_DIM, _HASH_VOCAB))
    hashes = _sc_hash(ids_flat)
    emb_pad = _sc_gather(hashes, embT, tail)
    W_pad = jnp.pad(W, ((0, 128 - _BIGRAM_DIM), (0, 0)))
    out = _tc_matmul(emb_pad, W_pad)
    return out.reshape(_B, _S, _MODEL_DIM)


# speedup vs baseline: 1.0973x; 1.0171x over previous
import jax
import jax.numpy as jnp
from jax import lax
from jax.experimental import pallas as pl
from jax.experimental.pallas import tpu as pltpu
from jax.experimental.pallas import tpu_sc as plsc

_HASH_VOCAB = 1000000
_BIGRAM_DIM = 64
_MODEL_DIM = 1024
_BOS_ID = 1

_NC = 2
_NS = 16
_NW = _NC * _NS
_LANES = 16

_B = 4
_S = 8192
_TOTAL = _B * _S
_CHUNK = _TOTAL // _NW

_VR = 31232
_SLAB = 256
_NSLAB = _VR // _SLAB
_TAIL_START = _HASH_VOCAB - 1024
_NTAIL = 4
_OUT_ROWS = _TOTAL + 128

_SENTINEL = 0x7F000000


def _hash_body(ids_hbm, out_hbm, ids_ext, hash_v, sem):
    del sem
    cid = lax.axis_index("c")
    sid = lax.axis_index("s")
    wid = cid * _NS + sid
    base = wid * _CHUNK

    ids_ext[pl.ds(0, _LANES)] = jnp.full((_LANES,), _BOS_ID, dtype=jnp.int32)
    pltpu.sync_copy(ids_hbm.at[pl.ds(base, _CHUNK)], ids_ext.at[pl.ds(8, _CHUNK)])

    @pl.when(lax.rem(wid, _S // _CHUNK) != 0)
    def _():
        pltpu.sync_copy(ids_hbm.at[pl.ds(base - 8, 8)], ids_ext.at[pl.ds(0, 8)])

    lanes = lax.iota(jnp.int32, _LANES)
    for j in range(8):
        for k in range(8):
            i = j * 8 + k
            cur = ids_ext[pl.ds(8 + i * _LANES, _LANES)]
            prev = plsc.load_gather(ids_ext, [lanes + (7 + i * _LANES)])
            x = (prev << 8) + (prev << 2) + (prev << 1) + prev + cur
            for c in (8000000, 4000000, 2000000, 1000000):
                x = jnp.where(x >= c, x - c, x)
            hash_v[j, pl.ds(k * _LANES, _LANES)] = x

    for j in range(8):
        pltpu.sync_copy(hash_v.at[j], out_hbm.at[pl.ds(base + j * 128, 128)])


def _sc_hash(ids_flat):
    mesh = plsc.VectorSubcoreMesh(core_axis_name="c", subcore_axis_name="s")
    return pl.kernel(
        _hash_body,
        out_type=jax.ShapeDtypeStruct((_TOTAL,), jnp.int32),
        mesh=mesh,
        scratch_types=[
            pltpu.VMEM((_CHUNK + 8,), jnp.int32),
            pltpu.VMEM((8, 128), jnp.int32),
            pltpu.SemaphoreType.DMA,
        ],
        compiler_params=pltpu.CompilerParams(needs_layout_passes=False),
    )(ids_flat)


def _gather_body(
    hash_hbm, embT_hbm, tail_hbm, out_hbm,
    hashchunk, binv, slablist, slab2d, stage, idxbuf, bkvec, sem0, sem1,
):
    cid = lax.axis_index("c")
    sid = lax.axis_index("s")
    wid = cid * _NS + sid
    v0 = wid * _VR
    lanes = lax.iota(jnp.int32, _LANES)
    is_last = wid == _NW - 1

    def _reset_idxbuf():
        for r in range(8):
            idxbuf[pl.ds(r * _LANES, _LANES)] = (
                lanes + (_TOTAL + r * _LANES)
            )

    _reset_idxbuf()

    def zrow(r, c):
        rv = jnp.full((_LANES,), r, jnp.int32)
        for g in range(4):
            plsc.store_scatter(
                stage,
                [rv, 64 + g * _LANES + lanes],
                jnp.zeros((_LANES,), jnp.float32),
            )
        return c

    lax.fori_loop(0, 128, zrow, 0)

    limit = jnp.where(is_last, jnp.int32(1 << 20), jnp.int32(_VR))

    def chunk_body(c, cnt):
        pltpu.sync_copy(hash_hbm.at[pl.ds(c * 4096, 4096)], hashchunk)

        def vreg_body(j, cnt):
            h = hashchunk[pl.ds(j * _LANES, _LANES)]
            pos = lanes + (c * 4096 + j * _LANES)
            hl = h - v0
            mask = (hl >= 0) & (hl < limit)
            key = (hl << 15) | pos
            plsc.store_compressed(binv.at[pl.ds(cnt, _LANES)], key, mask=mask)
            return cnt + jnp.sum(mask.astype(jnp.int32))

        return lax.fori_loop(0, 4096 // _LANES, vreg_body, cnt)

    cnt = lax.fori_loop(0, _TOTAL // 4096, chunk_body, 0)
    plsc.store_compressed(
        binv.at[pl.ds(cnt, _LANES)],
        jnp.full((_LANES,), _SENTINEL, jnp.int32),
        mask=jnp.ones((_LANES,), jnp.bool_),
    )
    nv = (cnt + _LANES - 1) // _LANES

    _NBK = 8

    starts = [jnp.int32(0)]
    for i in range(_NBK):

        def bpass(v, cur, _i=i):
            k = binv[pl.ds(v * _LANES, _LANES)]
            m = (k >> 27) == _i
            plsc.store_compressed(
                slablist.at[pl.ds(cur, _LANES)], k, mask=m
            )
            return cur + jnp.sum(m.astype(jnp.int32))

        starts.append(lax.fori_loop(0, nv, bpass, starts[i]))
    plsc.store_compressed(
        slablist.at[pl.ds(cnt, _LANES)],
        jnp.full((_LANES,), _SENTINEL, jnp.int32),
        mask=jnp.ones((_LANES,), jnp.bool_),
    )
    bv = jnp.zeros((_LANES,), jnp.int32)
    for i in range(_NBK + 1):
        bv = jnp.where(lanes == i, starts[i], bv)
    bkvec[pl.ds(0, _LANES)] = bv

    nslab = jnp.where(is_last, jnp.int32(_NSLAB + _NTAIL), jnp.int32(_NSLAB))

    def fire(s, sem, slot):
        @pl.when(s < _NSLAB)
        def _():
            pltpu.async_copy(
                embT_hbm.at[:, pl.ds(v0 + s * _SLAB, _SLAB)],
                slab2d.at[pl.ds(slot * _BIGRAM_DIM, _BIGRAM_DIM), :],
                sem,
            )

        @pl.when(s >= _NSLAB)
        def _():
            pltpu.async_copy(
                tail_hbm.at[:, pl.ds((s - _NSLAB) * _SLAB, _SLAB)],
                slab2d.at[pl.ds(slot * _BIGRAM_DIM, _BIGRAM_DIM), :],
                sem,
            )

    def wait(sem, slot):
        pltpu.make_async_copy(
            embT_hbm.at[:, pl.ds(0, _SLAB)],
            slab2d.at[pl.ds(slot * _BIGRAM_DIM, _BIGRAM_DIM), :],
            sem,
        ).wait()

    def process(s, slot, tot):
        hl0 = jnp.where(
            s < _NSLAB,
            s * _SLAB,
            (_TAIL_START - v0) + (s - _NSLAB) * _SLAB,
        ).astype(jnp.int32)
        lo = hl0 << 15
        hi = (hl0 + _SLAB) << 15
        bkt = hl0 >> 12
        bvv = bkvec[pl.ds(0, _LANES)]
        start_b = jnp.sum(jnp.where(lanes == bkt, bvv, 0))
        end_b = jnp.sum(jnp.where(lanes == bkt + 1, bvv, 0))

        v0r = start_b // _LANES
        nvr = (end_b + _LANES - 1) // _LANES - v0r

        def scan(i, mcnt):
            k = slablist[pl.ds((v0r + i) * _LANES, _LANES)]
            m = (k >= lo) & (k < hi)
            plsc.store_compressed(binv.at[pl.ds(mcnt, _LANES)], k, mask=m)
            return mcnt + jnp.sum(m.astype(jnp.int32))

        mcnt = lax.fori_loop(0, nvr, scan, 0)

        def ex(m, tot):
            mb = (m // _LANES) * _LANES
            kv = binv[pl.ds(mb, _LANES)]
            key_v = jnp.take(
                kv,
                jnp.full((_LANES,), m - mb, jnp.int32),
                mode="fill",
            )
            col_v = (key_v >> 15) - hl0
            p_v = key_v & 32767
            r = lax.rem(tot, 128)
            rv = jnp.full((_LANES,), r, jnp.int32)
            for g in range(4):
                vals = plsc.load_gather(
                    slab2d,
                    [slot * _BIGRAM_DIM + g * _LANES + lanes, col_v],
                )
                plsc.store_scatter(stage, [rv, g * _LANES + lanes], vals)
            plsc.store_scatter(idxbuf, [rv], p_v, mask=lanes == 0)

            @pl.when(r == 127)
            def _():
                pltpu.sync_copy(stage, out_hbm.at[idxbuf])
                _reset_idxbuf()

            return tot + 1

        return lax.fori_loop(0, mcnt, ex, tot)

    fire(0, sem0, 0)

    def pair(i, tot):
        s0 = 2 * i
        fire(s0 + 1, sem1, 1)
        wait(sem0, 0)
        tot = process(s0, 0, tot)

        @pl.when(s0 + 2 < nslab)
        def _():
            fire(s0 + 2, sem0, 0)

        wait(sem1, 1)
        return process(s0 + 1, 1, tot)

    tot = lax.fori_loop(0, nslab // 2, pair, 0)

    @pl.when(lax.rem(tot, 128) != 0)
    def _():
        pltpu.sync_copy(stage, out_hbm.at[idxbuf])


def _sc_gather(hashes, embT, tail):
    mesh = plsc.VectorSubcoreMesh(core_axis_name="c", subcore_axis_name="s")
    return pl.kernel(
        _gather_body,
        out_type=jax.ShapeDtypeStruct((_OUT_ROWS, 128), jnp.float32),
        mesh=mesh,
        scratch_types=[
            pltpu.VMEM((4096,), jnp.int32),
            pltpu.VMEM((_TOTAL + _LANES,), jnp.int32),
            pltpu.VMEM((_TOTAL + _LANES,), jnp.int32),
            pltpu.VMEM((2 * _BIGRAM_DIM, _SLAB), jnp.float32),
            pltpu.VMEM((128, 128), jnp.float32),
            pltpu.VMEM((128,), jnp.int32),
            pltpu.VMEM((_LANES,), jnp.int32),
            pltpu.SemaphoreType.DMA,
            pltpu.SemaphoreType.DMA,
        ],
        compiler_params=pltpu.CompilerParams(needs_layout_passes=False),
    )(hashes, embT, tail)


_MM_ROWS = 2048


def _mm_body(emb_ref, w_ref, out_ref):
    out_ref[...] = jnp.dot(
        emb_ref[...], w_ref[...], preferred_element_type=jnp.float32
    )


def _tc_matmul(emb_pad, W_pad):
    return pl.pallas_call(
        _mm_body,
        grid=(_TOTAL // _MM_ROWS,),
        in_specs=[
            pl.BlockSpec((_MM_ROWS, 128), lambda i: (i, 0)),
            pl.BlockSpec((128, _MODEL_DIM), lambda i: (0, 0)),
        ],
        out_specs=pl.BlockSpec((_MM_ROWS, _MODEL_DIM), lambda i: (i, 0)),
        out_shape=jax.ShapeDtypeStruct((_TOTAL, _MODEL_DIM), jnp.float32),
    )(emb_pad, W_pad)


def kernel(ids, embed, W):
    ids_flat = ids.reshape(_TOTAL).astype(jnp.int32)
    embT = embed.T
    tail = lax.slice(embT, (0, _TAIL_START), (_BIGRAM_DIM, _HASH_VOCAB))
    hashes = _sc_hash(ids_flat)
    emb_pad = _sc_gather(hashes, embT, tail)
    W_pad = jnp.pad(W, ((0, 128 - _BIGRAM_DIM), (0, 0)))
    out = _tc_matmul(emb_pad, W_pad)
    return out.reshape(_B, _S, _MODEL_DIM)
